# baseline (speedup 1.0000x reference)
import functools

import jax
import jax.numpy as jnp
from jax.experimental import pallas as pl
from jax.experimental.pallas import tpu as pltpu
from jax.experimental.pallas import tpu_sc as plsc

B = 16
H = 224
W = 224
R = 14
NSEG = R * R
NP = 208
NT = B * NP
CH = 192
INF = 3.0e38

_DNT = (((1,), (1,)), ((), ()))
_DN0 = (((0,), (0,)), ((), ()))

_dot = functools.partial(jnp.dot, preferred_element_type=jnp.float32)
_dot_hi = functools.partial(jnp.dot, preferred_element_type=jnp.float32,
                            precision=jax.lax.Precision.HIGHEST)


def _dg(a, b, dn):
    return jax.lax.dot_general(a, b, dn, preferred_element_type=jnp.float32)


def _split_hi_lo(v):
    hi = v.astype(jnp.bfloat16).astype(jnp.float32)
    return hi, v - hi


def _dotT_exact(a, b):
    b_hi, b_lo = _split_hi_lo(b)
    return _dg(a, b_hi, _DN0) + _dg(a, b_lo, _DN0)


def _conv(x, p, act):
    wref, gref, bref, beref = p
    g = gref[...]
    y = _dg(x, wref[...], _DNT) * g + (bref[...] * g + beref[...])
    return jax.nn.relu(y) if act else y


def _slic_sc():
    mesh = plsc.VectorSubcoreMesh(core_axis_name="c", subcore_axis_name="s")

    @functools.partial(
        pl.kernel, mesh=mesh,
        out_type=jax.ShapeDtypeStruct((B, H, W), jnp.int32),
        scratch_types=[pltpu.VMEM((W,), jnp.int32)])
    def k(out_hbm, row_v):
        wid = jax.lax.axis_index("s") * 2 + jax.lax.axis_index("c")
        img = wid // 2
        half = jax.lax.rem(wid, 2) * (R // 2)
        base = jax.lax.rem(wid, 2) * (H // 2)
        for rb in range(R // 2):
            rbeff = half + rb
            for m in range(R):
                row_v[pl.ds(m * 16, 16)] = (jnp.full((16,), m, jnp.int32)
                                            + rbeff * R)
            for r in range(16):
                pltpu.sync_copy(row_v, out_hbm.at[img, base + rb * 16 + r, :])

    return k()


def _stem_kernel(x_ref, feat_ref):
    x = x_ref[0]

    r224 = jax.lax.broadcasted_iota(jnp.int32, (H, R), 0)
    c14 = jax.lax.broadcasted_iota(jnp.int32, (H, R), 1)
    P = (r224 // 16 == c14).astype(jnp.float32)
    C = (r224 == 16 * c14 + 7).astype(jnp.float32)

    gy = jax.lax.broadcasted_iota(jnp.int32, (R, R), 0).astype(jnp.float32)
    gx = jax.lax.broadcasted_iota(jnp.int32, (R, R), 1).astype(jnp.float32)
    feat_ref[0, 0] = 16.0 * gy + 7.5
    feat_ref[0, 1] = 16.0 * gx + 7.5

    for c in range(3):
        X = x[c]
        Xr = X.reshape(R, 16, W)
        S1r = jnp.sum(Xr, axis=1)
        S2r = jnp.sum(Xr * Xr, axis=1)
        ctr_rows = jax.lax.slice_in_dim(Xr, 7, 8, axis=1).reshape(R, W)
        S1 = _dot_hi(S1r, P)
        S2 = _dot_hi(S2r, P)
        CTR = _dot_hi(ctr_rows, C)
        mean = S1 * (1.0 / 256.0)
        var = (S2 - 256.0 * mean * mean) * (1.0 / 255.0)
        std = jnp.sqrt(jnp.maximum(var, 0.0))
        feat_ref[0, 2 + c] = mean
        feat_ref[0, 5 + c] = std
        feat_ref[0, 8 + c] = CTR

    for c in range(11, 16):
        feat_ref[0, c] = jnp.zeros((R, R), jnp.float32)


def _stem_pallas(inputs):
    return pl.pallas_call(
        _stem_kernel,
        grid=(B,),
        in_specs=[pl.BlockSpec((1, 3, H, W), lambda i: (i, 0, 0, 0))],
        out_specs=pl.BlockSpec((1, 16, R, R), lambda i: (i, 0, 0, 0)),
        out_shape=jax.ShapeDtypeStruct((B, 16, R, R), jnp.float32),
    )(inputs)


def _grapher_block(x, k, dist_ref, maxf_ref, fc1, mr, fc2, ffn1, ffn2):
    y = _conv(x, fc1, act=False)

    row_np = jax.lax.broadcasted_iota(jnp.int32, (NP, NP), 0)
    lane_np = jax.lax.broadcasted_iota(jnp.int32, (NP, NP), 1)
    eye = (row_np == lane_np).astype(jnp.float32)
    for i in range(B):
        yi = y[i * NP:(i + 1) * NP, :]
        G = _dg(yi, yi, _DNT)
        sq_col = jnp.sum(yi * yi, axis=1, keepdims=True)
        sq_row = jnp.sum(eye * sq_col, axis=0, keepdims=True)
        d = (sq_col + sq_row) - 2.0 * G
        d = jnp.where(row_np[:, 0:1] >= NSEG, INF, d)
        dist_ref[:, i * NP:(i + 1) * NP] = d

    maxf_ref[...] = jnp.full((NT, CH), -INF, jnp.float32)
    sub_all = jax.lax.broadcasted_iota(jnp.int32, (NP, NT), 0)
    y_hi, y_lo = _split_hi_lo(y)

    def body(_, carry):
        d = dist_ref[...]
        minv = jnp.min(d, axis=0, keepdims=True)
        cand = jnp.where(d == minv, sub_all, 1000)
        jsel = jnp.min(cand, axis=0, keepdims=True)
        oh = sub_all == jsel
        dist_ref[...] = jnp.where(oh, INF, d)
        ohf = oh.astype(jnp.float32)
        for i in range(B):
            sl = slice(i * NP, (i + 1) * NP)
            ohi = ohf[:, sl]
            sel = _dg(ohi, y_hi[sl, :], _DN0) + _dg(ohi, y_lo[sl, :], _DN0)
            maxf_ref[sl, :] = jnp.maximum(maxf_ref[sl, :], sel)
        return carry

    jax.lax.fori_loop(0, k, body, 0)

    rel = maxf_ref[...] - y
    wmr, gmr, bmr, bemr = mr
    wmrv = wmr[...]
    gv = gmr[...]
    y = jax.nn.relu((_dg(y, wmrv[:, :CH], _DNT) + _dg(rel, wmrv[:, CH:], _DNT))
                    * gv + (bmr[...] * gv + bemr[...]))
    x = _conv(y, fc2, act=False) + x
    h = _conv(x, ffn1, act=True)
    return _conv(h, ffn2, act=False) + x


def _store_fmT(x, eye_np, fmt_ref):
    for i in range(B):
        xT = _dotT_exact(x[i * NP:(i + 1) * NP, :], eye_np)
        fmt_ref[i] = xT[:, :NSEG]


def _pc2_kernel(feats_ref, *refs):
    it = iter(refs)

    def take(n):
        return tuple(next(it) for _ in range(n))

    stem = [take(4) for _ in range(5)]
    blk1 = [take(4) for _ in range(5)]
    blk2 = [take(4) for _ in range(5)]
    pred1 = take(4)
    w2ref, b2ref = take(2)
    fm1t_ref, fm2t_ref, pred_ref, dist_ref, maxf_ref = take(5)

    x = feats_ref[...]
    for li, p in enumerate(stem):
        x = _conv(x, p, act=(li < 4))

    row_np = jax.lax.broadcasted_iota(jnp.int32, (NP, NP), 0)
    lane_np = jax.lax.broadcasted_iota(jnp.int32, (NP, NP), 1)
    eye_np = (row_np == lane_np).astype(jnp.float32)

    x = _grapher_block(x, 9, dist_ref, maxf_ref, *blk1)
    _store_fmT(x, eye_np, fm1t_ref)
    x = _grapher_block(x, 18, dist_ref, maxf_ref, *blk2)
    _store_fmT(x, eye_np, fm2t_ref)

    srow = jax.lax.broadcasted_iota(jnp.int32, (NT, B), 1)
    scol = jax.lax.broadcasted_iota(jnp.int32, (NT, B), 0)
    S = ((scol // NP == srow) & (scol - (scol // NP) * NP < NSEG)).astype(
        jnp.float32)
    pooled = _dotT_exact(S, x) * (1.0 / NSEG)
    h = _conv(pooled, pred1, act=True)
    pred_ref[...] = _dg(h, w2ref[...], _DNT) + b2ref[...]


def kernel(inputs, params, originalInput):
    x_slic = _slic_sc()
    feat = _stem_pallas(inputs)
    f = feat.reshape(B, 16, NSEG).transpose(0, 2, 1)
    f = jnp.pad(f, ((0, 0), (0, NP - NSEG), (0, 0))).reshape(NT, 16)

    def conv_args(p, pad_in=None):
        w = p['W']
        if pad_in is not None and pad_in != w.shape[1]:
            w = jnp.pad(w, ((0, 0), (0, pad_in - w.shape[1])))
        return [w, p['g'][None, :], p['b'][None, :], p['be'][None, :]]

    args = []
    for i, p in enumerate(params['stem']):
        args += conv_args(p, pad_in=16 if i == 0 else None)
    for blk in params['blocks']:
        for name in ('fc1', 'mr', 'fc2', 'ffn1', 'ffn2'):
            args += conv_args(blk[name])
    args += conv_args(params['pred1'])
    args += [params['pred2']['W'], params['pred2']['b'][None, :]]

    fm1t, fm2t, pred = pl.pallas_call(
        _pc2_kernel,
        out_shape=[jax.ShapeDtypeStruct((B, CH, NSEG), jnp.float32),
                   jax.ShapeDtypeStruct((B, CH, NSEG), jnp.float32),
                   jax.ShapeDtypeStruct((B, 1000), jnp.float32)],
        scratch_shapes=[pltpu.VMEM((NP, NT), jnp.float32),
                        pltpu.VMEM((NT, CH), jnp.float32)],
    )(f, *args)

    return (pred, x_slic,
            (fm1t.reshape(B, CH, R, R), fm2t.reshape(B, CH, R, R)))
